# Initial kernel scaffold; baseline (speedup 1.0000x reference)
import jax
import jax.numpy as jnp
from jax.experimental import pallas as pl


def kernel(hn0, hn1, Wself, Wmsg, Wup, bup, edge_index0, edge_index1, idx1):
    raise NotImplementedError("write your pallas kernel here")


# speedup vs baseline: 62.2058x; 62.2058x over previous
import dataclasses
import functools

import jax
import jax.numpy as jnp
from jax import lax
from jax.experimental import pallas as pl
from jax.experimental.pallas import tpu as pltpu
from jax.experimental.pallas import tpu_sc as plsc

D = 128
N0, N1 = 10000, 2500
E0, E1 = 320000, 80000
L = 2

NTILES = 32
CHUNK = 128

NP0 = 10112
NP1 = 2560
EP0 = 323584
EP1 = 81920
CPT0 = EP0 // (NTILES * CHUNK)
CPT1 = EP1 // (NTILES * CHUNK)
NCH0 = NP0 // CHUNK


def _vector_mesh():
    return plsc.VectorSubcoreMesh(core_axis_name="c", subcore_axis_name="s")


def _i32(x):
    return jnp.int32(x)


def _sc_compiler_params():
    cp = pltpu.CompilerParams()
    if "needs_layout_passes" in pltpu.CompilerParams.__dataclass_fields__:
        cp = dataclasses.replace(cp, needs_layout_passes=False)
    return cp


def _loop(n, body):
    def wrapped(i, carry):
        body(i)
        return carry

    lax.fori_loop(_i32(0), _i32(n), wrapped, None)


def _ln(h):
    mu = jnp.mean(h, axis=-1, keepdims=True)
    var = jnp.mean((h - mu) ** 2, axis=-1, keepdims=True)
    return (h - mu) * lax.rsqrt(var + 1e-5)


def _ln_body(h_ref, o_ref):
    o_ref[...] = _ln(h_ref[...])


def tc_ln(h):
    return pl.pallas_call(
        _ln_body,
        out_shape=jax.ShapeDtypeStruct(h.shape, jnp.float32),
    )(h)


def _ms_body(h_ref, wm_ref, ws_ref, m_ref, s_ref):
    h = h_ref[...]
    m_ref[...] = jnp.dot(h, wm_ref[...], preferred_element_type=jnp.float32, precision=lax.Precision.HIGHEST)
    s_ref[...] = jnp.dot(h, ws_ref[...], preferred_element_type=jnp.float32, precision=lax.Precision.HIGHEST)


def tc_ms(h, wm, ws):
    n = h.shape[0]
    return pl.pallas_call(
        _ms_body,
        out_shape=[jax.ShapeDtypeStruct((n, D), jnp.float32),
                   jax.ShapeDtypeStruct((n, D), jnp.float32)],
    )(h, wm, ws)


def _comb1_body(s_ref, a_ref, b_ref, wt_ref, u_ref, hn_ref):
    h = s_ref[...] + a_ref[...] + b_ref[...]
    u_ref[...] = jnp.dot(h, wt_ref[...], preferred_element_type=jnp.float32, precision=lax.Precision.HIGHEST)
    hn_ref[...] = _ln(h)


def tc_comb1(s, aggA, aggB, wtop):
    n = s.shape[0]
    return pl.pallas_call(
        _comb1_body,
        out_shape=[jax.ShapeDtypeStruct((n, D), jnp.float32),
                   jax.ShapeDtypeStruct((n, D), jnp.float32)],
    )(s, aggA, aggB, wtop)


def _comb0_body(s_ref, a_ref, b_ref, wb_ref, bias_ref, o_ref):
    h = s_ref[...] + a_ref[...] + b_ref[...]
    o_ref[...] = (h + jnp.dot(h, wb_ref[...], preferred_element_type=jnp.float32, precision=lax.Precision.HIGHEST)
                  + bias_ref[...])


def tc_comb0(s, aggA, aggB, wbot, bias_row):
    n = s.shape[0]
    return pl.pallas_call(
        _comb0_body,
        out_shape=jax.ShapeDtypeStruct((n, D), jnp.float32),
    )(s, aggA, aggB, wbot, bias_row)


def _fin0_body(h_ref, inp_ref, o_ref):
    o_ref[...] = _ln(h_ref[...] + inp_ref[...])


def tc_fin0(h0c, inp):
    return pl.pallas_call(
        _fin0_body,
        out_shape=jax.ShapeDtypeStruct(h0c.shape, jnp.float32),
    )(h0c, inp)


def sc_segsum(msg, srcp, dstp, zeros_np, NP, cpt):
    stripe = NP // 16
    ep_half = (cpt * NTILES * CHUNK) // 2

    @functools.partial(
        pl.kernel,
        out_type=jax.ShapeDtypeStruct((2, NP, D), jnp.float32),
        mesh=_vector_mesh(),
        scratch_types=[
            pltpu.VMEM((CHUNK,), jnp.int32),
            pltpu.VMEM((CHUNK,), jnp.int32),
            pltpu.VMEM((CHUNK, D), jnp.float32),
            pltpu.VMEM_SHARED((NP, D), jnp.float32),
            pltpu.SemaphoreType.DMA,
        ],
    )
    def k(msg_hbm, src_hbm, dst_hbm, z_hbm, out_hbm,
          src_v, dst_v, rows_v, acc_sh, sem):
        cid = lax.axis_index("c")
        sid = lax.axis_index("s")
        off = sid * _i32(stripe)
        pltpu.sync_copy(z_hbm.at[pl.ds(off, stripe)],
                        acc_sh.at[pl.ds(off, stripe)])
        plsc.subcore_barrier()

        tile_base = cid * _i32(ep_half) + sid * _i32(cpt * CHUNK)

        def chunk(kk):
            base = tile_base + kk * _i32(CHUNK)
            pltpu.sync_copy(src_hbm.at[pl.ds(base, CHUNK)], src_v)
            pltpu.sync_copy(dst_hbm.at[pl.ds(base, CHUNK)], dst_v)
            pltpu.async_copy(msg_hbm.at[src_v], rows_v, sem).wait()
            pltpu.sync_copy(rows_v, acc_sh.at[dst_v], add=True)

        _loop(cpt, chunk)

        plsc.subcore_barrier()
        pltpu.sync_copy(acc_sh.at[pl.ds(off, stripe)],
                        out_hbm.at[cid, pl.ds(off, stripe)])

    return k(msg, srcp, dstp, zeros_np)


NIP = 2512


def sc_winmap(idx1_pad):
    @functools.partial(
        pl.kernel,
        out_type=jax.ShapeDtypeStruct((NP0,), jnp.int32),
        mesh=_vector_mesh(),
        scratch_types=[
            pltpu.VMEM((NIP,), jnp.int32),
            pltpu.VMEM((NP0,), jnp.int32),
        ],
        compiler_params=_sc_compiler_params(),
    )
    def k(idx_hbm, win_hbm, idx_v, win_v):
        cid = lax.axis_index("c")
        sid = lax.axis_index("s")

        @pl.when(jnp.logical_and(cid == _i32(0), sid == _i32(0)))
        def _():
            pltpu.sync_copy(idx_hbm, idx_v)

            def init(t):
                win_v[pl.ds(t * _i32(16), 16)] = jnp.full(
                    (16,), N1, jnp.int32)

            _loop(NP0 // 16, init)

            lanes = lax.iota(jnp.int32, 16)

            def group(t):
                j0 = t * _i32(16)
                idxvec = idx_v[pl.ds(j0, 16)]
                jvec = lanes + j0
                for lane in range(16):
                    plsc.store_scatter(win_v, [idxvec], jvec,
                                       mask=lanes == _i32(lane))

            _loop(NIP // 16, group)

            pltpu.sync_copy(win_v, win_hbm)

    return k(idx1_pad)


def sc_upgather(u1pad, win):
    @functools.partial(
        pl.kernel,
        out_type=jax.ShapeDtypeStruct((NP0, D), jnp.float32),
        mesh=_vector_mesh(),
        scratch_types=[
            pltpu.VMEM((CHUNK,), jnp.int32),
            pltpu.VMEM((CHUNK, D), jnp.float32),
            pltpu.SemaphoreType.DMA,
        ],
    )
    def k(u_hbm, win_hbm, out_hbm, widx_v, rows_v, sem):
        cid = lax.axis_index("c")
        sid = lax.axis_index("s")
        wid = cid * _i32(16) + sid
        for kk in range(3):
            ch = wid + _i32(kk * NTILES)

            @pl.when(ch < _i32(NCH0))
            def _():
                base = ch * _i32(CHUNK)
                pltpu.sync_copy(win_hbm.at[pl.ds(base, CHUNK)], widx_v)
                pltpu.async_copy(u_hbm.at[widx_v], rows_v, sem).wait()
                pltpu.sync_copy(rows_v, out_hbm.at[pl.ds(base, CHUNK)])

    return k(u1pad, win)


def kernel(hn0, hn1, Wself, Wmsg, Wup, bup, edge_index0, edge_index1, idx1):
    f32 = jnp.float32
    i32 = jnp.int32

    h0 = jnp.pad(hn0.astype(f32), ((0, NP0 - N0), (0, 0)))
    h1 = jnp.pad(hn1.astype(f32), ((0, NP1 - N1), (0, 0)))

    src0 = jnp.pad(edge_index0[0].astype(i32), (0, EP0 - E0))
    dst0 = jnp.pad(edge_index0[1].astype(i32), (0, EP0 - E0),
                   constant_values=NP0 - 1)
    src1 = jnp.pad(edge_index1[0].astype(i32), (0, EP1 - E1))
    dst1 = jnp.pad(edge_index1[1].astype(i32), (0, EP1 - E1),
                   constant_values=NP1 - 1)
    idx1_pad = jnp.pad(idx1.astype(i32), (0, NIP - N1), constant_values=N0)

    z0 = jnp.zeros((NP0, D), f32)
    z1 = jnp.zeros((NP1, D), f32)

    Wself = Wself.astype(f32)
    Wmsg = Wmsg.astype(f32)
    Wup = Wup.astype(f32)
    bup = bup.astype(f32)

    h0 = tc_ln(h0)
    h1 = tc_ln(h1)
    win = sc_winmap(idx1_pad)

    for l in range(L):
        m0, s0 = tc_ms(h0, Wmsg[l, 0], Wself[l, 0])
        agg0 = sc_segsum(m0, src0, dst0, z0, NP0, CPT0)
        m1, s1 = tc_ms(h1, Wmsg[l, 1], Wself[l, 1])
        agg1 = sc_segsum(m1, src1, dst1, z1, NP1, CPT1)

        u1, h1 = tc_comb1(s1, agg1[0], agg1[1], Wup[l, :D])
        h0c = tc_comb0(s0, agg0[0], agg0[1], Wup[l, D:],
                       jnp.reshape(bup[l], (1, D)))
        inp = sc_upgather(u1, win)
        h0 = tc_fin0(h0c, inp)

    return (h0[:N0].astype(jnp.float64), h1[:N1].astype(jnp.float64))
